# baseline (speedup 1.0000x reference)
import functools

import jax
import jax.numpy as jnp
from jax import lax
from jax.experimental import pallas as pl
from jax.experimental.pallas import tpu as pltpu
from jax.experimental.pallas import tpu_sc as plsc

_N = 10000
_E = 160000
_DIM = 16
_B = 64

_NC = 2
_NS = 16
_NW = _NC * _NS

_CH = 128
_E_PAD = 163840
_EPW = _E_PAD // _NW
_NCHUNK = _EPW // _CH
_N_PAD = 10240
_NPT = _N_PAD // _NS

_f32 = jnp.float32


_FD = 20


def _sc_gather_body(tbl_hbm, src2_hbm, xj_out, idx_v, rows_v, sem):
    c = lax.axis_index("c")
    s = lax.axis_index("s")
    wid = s * _NC + c
    base = wid * _EPW
    rbase = wid * _NCHUNK
    pltpu.sync_copy(src2_hbm.at[pl.ds(rbase, _NCHUNK)], idx_v)

    def fire_drain(o, carry):
        ds = []
        for jj in range(_FD):
            j = o * _FD + jj
            ds.append(pltpu.async_copy(
                tbl_hbm.at[idx_v.at[j]],
                rows_v.at[pl.ds(j * _CH, _CH)], sem))
        for dsc in ds:
            dsc.wait()
        return carry

    lax.fori_loop(0, _NCHUNK // _FD, fire_drain, 0)
    pltpu.sync_copy(rows_v, xj_out.at[pl.ds(base, _EPW)])


@functools.lru_cache(maxsize=None)
def _get_sc_gather():
    return pl.kernel(
        _sc_gather_body,
        out_type=jax.ShapeDtypeStruct((_E_PAD, _DIM), _f32),
        mesh=plsc.VectorSubcoreMesh(core_axis_name="c", subcore_axis_name="s",
                                    num_cores=_NC, num_subcores=_NS),
        scratch_types=[
            pltpu.VMEM((_NCHUNK, _CH), jnp.int32),
            pltpu.VMEM((_EPW, _DIM), _f32),
            pltpu.SemaphoreType.DMA,
        ],
        compiler_params=pltpu.CompilerParams(use_tc_tiling_on_sc=False),
    )


def _sc_scatter_body(msg_hbm, dst2_hbm, zero_hbm, agg_out, agg_sh, idx_v,
                     msg_v, sem):
    c = lax.axis_index("c")
    s = lax.axis_index("s")
    pltpu.sync_copy(zero_hbm.at[pl.ds(s * _NPT, _NPT)],
                    agg_sh.at[pl.ds(s * _NPT, _NPT)])
    wid = s * _NC + c
    base = wid * _EPW
    rbase = wid * _NCHUNK
    pltpu.sync_copy(dst2_hbm.at[pl.ds(rbase, _NCHUNK)], idx_v)
    pltpu.sync_copy(msg_hbm.at[pl.ds(base, _EPW)], msg_v)
    plsc.subcore_barrier()

    def fire_drain(o, carry):
        ds = []
        for jj in range(_FD):
            j = o * _FD + jj
            ds.append(pltpu.async_copy(
                msg_v.at[pl.ds(j * _CH, _CH)],
                agg_sh.at[idx_v.at[j]], sem, add=True))
        for dsc in ds:
            dsc.wait()
        return carry

    lax.fori_loop(0, _NCHUNK // _FD, fire_drain, 0)
    plsc.subcore_barrier()
    pltpu.sync_copy(agg_sh.at[pl.ds(s * _NPT, _NPT)],
                    agg_out.at[c, pl.ds(s * _NPT, _NPT)])


@functools.lru_cache(maxsize=None)
def _get_sc_scatter():
    return pl.kernel(
        _sc_scatter_body,
        out_type=jax.ShapeDtypeStruct((_NC, _N_PAD, _DIM), _f32),
        mesh=plsc.VectorSubcoreMesh(core_axis_name="c", subcore_axis_name="s",
                                    num_cores=_NC, num_subcores=_NS),
        scratch_types=[
            pltpu.VMEM_SHARED((_N_PAD, _DIM), _f32),
            pltpu.VMEM((_NCHUNK, _CH), jnp.int32),
            pltpu.VMEM((_EPW, _DIM), _f32),
            pltpu.SemaphoreType.DMA,
        ],
        compiler_params=pltpu.CompilerParams(use_tc_tiling_on_sc=False),
    )


_NBLK = 2000
_EBLK = 2048


def _pre_node_body(x_ref, w0t_ref, b0_ref, c0_ref, c1_ref, out_ref, cinv_ref):
    out_ref[...] = jax.nn.relu(
        jnp.dot(x_ref[...], w0t_ref[...], preferred_element_type=_f32)
        + b0_ref[...])
    cnt = c0_ref[...] + c1_ref[...]
    cinv_ref[...] = 1.0 / jnp.maximum(cnt, 1.0)


def _eh_body(ea_ref, w1bd_ref, be1_ref, eh_ref):
    eh_ref[...] = jax.nn.relu(
        jnp.dot(ea_ref[...], w1bd_ref[...], preferred_element_type=_f32)
        + be1_ref[...])


def _msg_body(xj_ref, eh_ref, w2bd_ref, be2v_ref, rbd_ref, msg_ref):
    xj = xj_ref[...]
    eh = eh_ref[...]
    w = jnp.dot(eh, w2bd_ref[...], preferred_element_type=_f32) + be2v_ref[...]
    wb = w.astype(jnp.bfloat16).astype(_f32)
    xjb = xj.astype(jnp.bfloat16).astype(_f32)
    xjx = jnp.dot(xjb, rbd_ref[...], preferred_element_type=_f32)
    p2 = xjx * wb
    acc = p2[:, 0:1024] + p2[:, 1024:2048]
    acc = acc[:, 0:512] + acc[:, 512:1024]
    acc = acc[:, 0:256] + acc[:, 256:512]
    msg_ref[...] = acc[:, 0:128] + acc[:, 128:256]


def _gru_body(a0_ref, a1_ref, cinv_ref, h_ref, root_ref, cb_ref,
              wih_ref, whh_ref, bih_ref, bhh_ref, out_ref):
    h = h_ref[...]
    agg = (a0_ref[...] + a1_ref[...]) * cinv_ref[...]
    m = jax.nn.relu(
        agg + jnp.dot(h, root_ref[...], preferred_element_type=_f32)
        + cb_ref[...])
    gi = jnp.dot(m, wih_ref[...], preferred_element_type=_f32) + bih_ref[...]
    gh = jnp.dot(h, whh_ref[...], preferred_element_type=_f32) + bhh_ref[...]
    r = jax.nn.sigmoid(gi[:, 0:16] + gh[:, 0:16])
    z = jax.nn.sigmoid(gi[:, 16:32] + gh[:, 16:32])
    nn = jnp.tanh(gi[:, 32:48] + r * gh[:, 32:48])
    out_ref[...] = (1.0 - z) * nn + z * h


def _s2s_body(out_ref, b_ref, sWihT_ref, sWhhT_ref, sbih_ref, sbhh_ref,
              mWihT_ref, mWhhT_ref, mbih_ref, mbhh_ref,
              W1T_ref, b1_ref, W3T_ref, b3_ref,
              v_ref, hm_ref, cm_ref):
    out = out_ref[...]
    bb = b_ref[...]
    gid = lax.broadcasted_iota(jnp.int32, (_N, _B), 1)
    oh = (bb == gid).astype(_f32)

    q_star = jnp.zeros((_B, 2 * _DIM), _f32)
    hs = jnp.zeros((_B, _DIM), _f32)
    cs = jnp.zeros((_B, _DIM), _f32)
    for _ in range(6):
        g = (jnp.dot(q_star, sWihT_ref[...], preferred_element_type=_f32)
             + sbih_ref[...]
             + jnp.dot(hs, sWhhT_ref[...], preferred_element_type=_f32)
             + sbhh_ref[...])
        ig, fg, gg, og = (g[:, 0:16], g[:, 16:32], g[:, 32:48], g[:, 48:64])
        cs = jax.nn.sigmoid(fg) * cs + jax.nn.sigmoid(ig) * jnp.tanh(gg)
        hs = jax.nn.sigmoid(og) * jnp.tanh(cs)

        qb = jnp.dot(oh, hs, preferred_element_type=_f32,
                     precision=lax.Precision.HIGHEST)
        escore = jnp.sum(out * qb, axis=1, keepdims=True)
        masked = jnp.where(oh > 0.0, escore, -jnp.inf)
        emax = jnp.max(masked, axis=0, keepdims=True)
        emax = jnp.where(emax > -jnp.inf, emax, 0.0)
        emaxb = jnp.sum(oh * emax, axis=1, keepdims=True)
        ee = jnp.exp(escore - emaxb)
        denom = jnp.sum(oh * ee, axis=0, keepdims=True)
        denomb = jnp.sum(oh * denom, axis=1, keepdims=True)
        a = ee / (denomb + 1e-16)
        r = jnp.einsum('ng,nf->gf', oh, a * out,
                       preferred_element_type=_f32,
                       precision=lax.Precision.HIGHEST)
        q_star = jnp.concatenate([hs, r], axis=1)

    g = (jnp.dot(q_star, mWihT_ref[...], preferred_element_type=_f32)
         + mbih_ref[...] + mbhh_ref[...])
    ig, fg, gg, og = (g[:, 0:16], g[:, 16:32], g[:, 32:48], g[:, 48:64])
    cm = jax.nn.sigmoid(ig) * jnp.tanh(gg)
    hm = jax.nn.sigmoid(og) * jnp.tanh(cm)
    o = jax.nn.relu(jnp.dot(hm, W1T_ref[...], preferred_element_type=_f32)
                    + b1_ref[...])
    v_ref[...] = jnp.dot(o, W3T_ref[...], preferred_element_type=_f32) + b3_ref[...]
    hm_ref[...] = hm
    cm_ref[...] = cm


def _full(shape):
    return pl.BlockSpec(shape, lambda i: (0,) * len(shape))


def _rows(blk, width):
    return pl.BlockSpec((blk, width), lambda i: (i, 0))


def _pre_node(xp, w0t, b0, c0, c1):
    return pl.pallas_call(
        _pre_node_body,
        grid=(_N // _NBLK,),
        in_specs=[_rows(_NBLK, 8), _full((8, _DIM)), _full((1, _DIM)),
                  _rows(_NBLK, _DIM), _rows(_NBLK, _DIM)],
        out_specs=[_rows(_NBLK, _DIM), _rows(_NBLK, _DIM)],
        out_shape=[jax.ShapeDtypeStruct((_N, _DIM), _f32),
                   jax.ShapeDtypeStruct((_N, _DIM), _f32)],
    )(xp, w0t, b0, c0, c1)


_EP8 = _E_PAD // 8
_EBLK8 = _EBLK // 8


def _tc_eh(eap, w1bd, be1v):
    return pl.pallas_call(
        _eh_body,
        grid=(_EP8 // _EBLK8,),
        in_specs=[_rows(_EBLK8, 128), _full((128, 128)), _full((1, 128))],
        out_specs=_rows(_EBLK8, 128),
        out_shape=jax.ShapeDtypeStruct((_EP8, 128), _f32),
    )(eap, w1bd, be1v)


def _tc_msg(xj, eh, w2bd, be2v, rbd):
    return pl.pallas_call(
        _msg_body,
        grid=(_EP8 // _EBLK8,),
        in_specs=[_rows(_EBLK8, 128), _rows(_EBLK8, 128),
                  _full((128, 2048)), _full((1, 2048)), _full((128, 2048))],
        out_specs=_rows(_EBLK8, 128),
        out_shape=jax.ShapeDtypeStruct((_EP8, 128), _f32),
    )(xj, eh, w2bd, be2v, rbd)


def _tc_gru(a0, a1, cinv, h, root, cb, wih, whh, bih, bhh):
    return pl.pallas_call(
        _gru_body,
        grid=(_N // _NBLK,),
        in_specs=[_rows(_NBLK, _DIM), _rows(_NBLK, _DIM), _rows(_NBLK, _DIM),
                  _rows(_NBLK, _DIM), _full((_DIM, _DIM)), _full((1, _DIM)),
                  _full((_DIM, 3 * _DIM)), _full((_DIM, 3 * _DIM)),
                  _full((1, 3 * _DIM)), _full((1, 3 * _DIM))],
        out_specs=_rows(_NBLK, _DIM),
        out_shape=jax.ShapeDtypeStruct((_N, _DIM), _f32),
    )(a0, a1, cinv, h, root, cb, wih, whh, bih, bhh)


def _tc_s2s(out, b2, sWihT, sWhhT, sbih, sbhh, mWihT, mWhhT, mbih, mbhh,
            W1T, b1, W3T, b3):
    return pl.pallas_call(
        _s2s_body,
        grid=(1,),
        in_specs=[_rows(_N, _DIM), _rows(_N, 1),
                  _full((2 * _DIM, 4 * _DIM)), _full((_DIM, 4 * _DIM)),
                  _full((1, 4 * _DIM)), _full((1, 4 * _DIM)),
                  _full((2 * _DIM, 4 * _DIM)), _full((_DIM, 4 * _DIM)),
                  _full((1, 4 * _DIM)), _full((1, 4 * _DIM)),
                  _full((_DIM, _DIM)), _full((1, _DIM)),
                  _full((_DIM, 1)), _full((1, 1))],
        out_specs=[_full((_B, 1)), _full((_B, _DIM)), _full((_B, _DIM))],
        out_shape=[jax.ShapeDtypeStruct((_B, 1), _f32),
                   jax.ShapeDtypeStruct((_B, _DIM), _f32),
                   jax.ShapeDtypeStruct((_B, _DIM), _f32)],
    )(out, b2, sWihT, sWhhT, sbih, sbhh, mWihT, mWhhT, mbih, mbhh,
      W1T, b1, W3T, b3)


def kernel(x, edge_index, edge_attr, batch, W0, b0, We1, be1, We2, be2, root,
           conv_bias, gWih, gWhh, gbih, gbhh, sWih, sWhh, sbih, sbhh,
           mWih, mWhh, mbih, mbhh, W1, b1, W3, b3):
    pad_e = _E_PAD - _E
    src = edge_index[0]
    dst = edge_index[1]
    srcp = jnp.concatenate(
        [src, jnp.zeros((pad_e,), jnp.int32)]).reshape(_E_PAD // _CH, _CH)
    dstp = jnp.concatenate(
        [dst, jnp.full((pad_e,), _N, jnp.int32)]).reshape(_E_PAD // _CH, _CH)
    eap = jnp.concatenate(
        [edge_attr, jnp.zeros((pad_e, edge_attr.shape[1]), _f32)], axis=0)
    xp = jnp.concatenate([x, jnp.zeros((_N, 8 - x.shape[1]), _f32)], axis=1)
    w0tp = jnp.concatenate(
        [W0.T, jnp.zeros((8 - x.shape[1], _DIM), _f32)], axis=0)
    zeros_tbl = jnp.zeros((_N_PAD, _DIM), _f32)
    ones_msg = jnp.ones((_E_PAD, _DIM), _f32)

    we1t = We1.T
    w1bd = jnp.kron(jnp.eye(8, dtype=_f32), we1t)
    be1v = jnp.tile(be1, 8).reshape(1, 128)
    we2t = We2.T
    L = jnp.arange(2048, dtype=jnp.int32)
    dL, jL, fL = L // 128, (L % 128) // _DIM, L % _DIM
    m = jnp.arange(128, dtype=jnp.int32)
    jm, cm = m // _DIM, m % _DIM
    w2bd = jnp.where(jm[:, None] == jL[None, :],
                     we2t[cm[:, None], (dL * _DIM + fL)[None, :]], 0.0)
    be2v = be2[dL * _DIM + fL].reshape(1, 2048)
    rbd = ((jm[:, None] == jL[None, :]) &
           (cm[:, None] == dL[None, :])).astype(_f32)

    sc_scatter = _get_sc_scatter()
    sc_gather = _get_sc_gather()

    cntp = sc_scatter(ones_msg, dstp, zeros_tbl)
    out, cinv = _pre_node(xp, w0tp, b0.reshape(1, _DIM),
                          cntp[0, :_N], cntp[1, :_N])
    eh = _tc_eh(eap.reshape(_EP8, 128), w1bd, be1v)

    for _ in range(6):
        xj = sc_gather(out, srcp)
        msgp = _tc_msg(xj.reshape(_EP8, 128), eh, w2bd, be2v, rbd)
        msg = msgp.reshape(_E_PAD, _DIM)
        aggp = sc_scatter(msg, dstp, zeros_tbl)
        out = _tc_gru(aggp[0, :_N], aggp[1, :_N], cinv, out, root,
                      conv_bias.reshape(1, _DIM), gWih.T, gWhh.T,
                      gbih.reshape(1, 3 * _DIM), gbhh.reshape(1, 3 * _DIM))

    v, hm, cm = _tc_s2s(out, batch.reshape(_N, 1), sWih.T, sWhh.T,
                        sbih.reshape(1, 4 * _DIM), sbhh.reshape(1, 4 * _DIM),
                        mWih.T, mWhh.T, mbih.reshape(1, 4 * _DIM),
                        mbhh.reshape(1, 4 * _DIM), W1.T, b1.reshape(1, _DIM),
                        W3.T, b3.reshape(1, 1))
    return v[None], hm[None], cm[None]


# speedup vs baseline: 1.4980x; 1.4980x over previous
import functools

import jax
import jax.numpy as jnp
from jax import lax
from jax.experimental import pallas as pl
from jax.experimental.pallas import tpu as pltpu
from jax.experimental.pallas import tpu_sc as plsc

_N = 10000
_E = 160000
_DIM = 16
_B = 64

_NC = 2
_NS = 16
_NW = _NC * _NS

_CH = 128
_E_PAD = 163840
_EPW = _E_PAD // _NW
_NCHUNK = _EPW // _CH
_N_PAD = 10240
_NPT = _N_PAD // _NS

_f32 = jnp.float32


_FD = 20


def _sc_gather_body(tbl_hbm, src2_hbm, xj_out, idx_v, rows_v, sem):
    c = lax.axis_index("c")
    s = lax.axis_index("s")
    wid = s * _NC + c
    base = wid * _EPW
    rbase = wid * _NCHUNK
    pltpu.sync_copy(src2_hbm.at[pl.ds(rbase, _NCHUNK)], idx_v)

    def fire_drain(o, carry):
        ds = []
        for jj in range(_FD):
            j = o * _FD + jj
            ds.append(pltpu.async_copy(
                tbl_hbm.at[idx_v.at[j]],
                rows_v.at[pl.ds(j * _CH, _CH)], sem))
        for dsc in ds:
            dsc.wait()
        return carry

    lax.fori_loop(0, _NCHUNK // _FD, fire_drain, 0)
    pltpu.sync_copy(rows_v, xj_out.at[pl.ds(base, _EPW)])


@functools.lru_cache(maxsize=None)
def _get_sc_gather():
    return pl.kernel(
        _sc_gather_body,
        out_type=jax.ShapeDtypeStruct((_E_PAD, _DIM), _f32),
        mesh=plsc.VectorSubcoreMesh(core_axis_name="c", subcore_axis_name="s",
                                    num_cores=_NC, num_subcores=_NS),
        scratch_types=[
            pltpu.VMEM((_NCHUNK, _CH), jnp.int32),
            pltpu.VMEM((_EPW, _DIM), _f32),
            pltpu.SemaphoreType.DMA,
        ],
        compiler_params=pltpu.CompilerParams(use_tc_tiling_on_sc=False),
    )


def _sc_scatter_body(msg_hbm, dst2_hbm, zero_hbm, agg_out, agg_sh, idx_v,
                     msg_v, sem):
    c = lax.axis_index("c")
    s = lax.axis_index("s")
    pltpu.sync_copy(zero_hbm.at[pl.ds(s * _NPT, _NPT)],
                    agg_sh.at[pl.ds(s * _NPT, _NPT)])
    wid = s * _NC + c
    base = wid * _EPW
    rbase = wid * _NCHUNK
    pltpu.sync_copy(dst2_hbm.at[pl.ds(rbase, _NCHUNK)], idx_v)
    pltpu.sync_copy(msg_hbm.at[pl.ds(base, _EPW)], msg_v)
    plsc.subcore_barrier()

    def fire_drain(o, carry):
        ds = []
        for jj in range(_FD):
            j = o * _FD + jj
            ds.append(pltpu.async_copy(
                msg_v.at[pl.ds(j * _CH, _CH)],
                agg_sh.at[idx_v.at[j]], sem, add=True))
        for dsc in ds:
            dsc.wait()
        return carry

    lax.fori_loop(0, _NCHUNK // _FD, fire_drain, 0)
    plsc.subcore_barrier()
    pltpu.sync_copy(agg_sh.at[pl.ds(s * _NPT, _NPT)],
                    agg_out.at[c, pl.ds(s * _NPT, _NPT)])


@functools.lru_cache(maxsize=None)
def _get_sc_scatter():
    return pl.kernel(
        _sc_scatter_body,
        out_type=jax.ShapeDtypeStruct((_NC, _N_PAD, _DIM), _f32),
        mesh=plsc.VectorSubcoreMesh(core_axis_name="c", subcore_axis_name="s",
                                    num_cores=_NC, num_subcores=_NS),
        scratch_types=[
            pltpu.VMEM_SHARED((_N_PAD, _DIM), _f32),
            pltpu.VMEM((_NCHUNK, _CH), jnp.int32),
            pltpu.VMEM((_EPW, _DIM), _f32),
            pltpu.SemaphoreType.DMA,
        ],
        compiler_params=pltpu.CompilerParams(use_tc_tiling_on_sc=False),
    )


_NBLK = 2000
_EBLK = 2048


def _pre_node_body(x_ref, w0t_ref, b0_ref, c0_ref, c1_ref, out_ref, cinv_ref):
    out_ref[...] = jax.nn.relu(
        jnp.dot(x_ref[...], w0t_ref[...], preferred_element_type=_f32)
        + b0_ref[...])
    cnt = c0_ref[...] + c1_ref[...]
    cinv_ref[...] = 1.0 / jnp.maximum(cnt, 1.0)


def _eh_body(ea_ref, we1t_ref, be1_ref, eh_ref):
    eh_ref[...] = jax.nn.relu(
        jnp.dot(ea_ref[...], we1t_ref[...], preferred_element_type=_f32)
        + be1_ref[...])


def _msg_body(xj_ref, eh_ref, we2t_ref, be2_ref, r_ref, s_ref, msg_ref):
    xj = xj_ref[...]
    eh = eh_ref[...]
    w = jnp.dot(eh, we2t_ref[...], preferred_element_type=_f32) + be2_ref[...]
    wb = w.astype(jnp.bfloat16).astype(_f32)
    xjb = xj.astype(jnp.bfloat16).astype(_f32)
    xjx = jnp.dot(xjb, r_ref[...], preferred_element_type=_f32)
    p2 = xjx * wb
    msg_ref[...] = jnp.dot(p2, s_ref[...], preferred_element_type=_f32,
                           precision=lax.Precision.HIGHEST)


def _gru_body(a0_ref, a1_ref, cinv_ref, h_ref, root_ref, cb_ref,
              wih_ref, whh_ref, bih_ref, bhh_ref, out_ref):
    h = h_ref[...]
    agg = (a0_ref[...] + a1_ref[...]) * cinv_ref[...]
    m = jax.nn.relu(
        agg + jnp.dot(h, root_ref[...], preferred_element_type=_f32)
        + cb_ref[...])
    gi = jnp.dot(m, wih_ref[...], preferred_element_type=_f32) + bih_ref[...]
    gh = jnp.dot(h, whh_ref[...], preferred_element_type=_f32) + bhh_ref[...]
    r = jax.nn.sigmoid(gi[:, 0:16] + gh[:, 0:16])
    z = jax.nn.sigmoid(gi[:, 16:32] + gh[:, 16:32])
    nn = jnp.tanh(gi[:, 32:48] + r * gh[:, 32:48])
    out_ref[...] = (1.0 - z) * nn + z * h


def _s2s_body(out_ref, b_ref, sWihT_ref, sWhhT_ref, sbih_ref, sbhh_ref,
              mWihT_ref, mWhhT_ref, mbih_ref, mbhh_ref,
              W1T_ref, b1_ref, W3T_ref, b3_ref,
              v_ref, hm_ref, cm_ref):
    out = out_ref[...]
    bb = b_ref[...]
    gid = lax.broadcasted_iota(jnp.int32, (_N, _B), 1)
    oh = (bb == gid).astype(_f32)

    q_star = jnp.zeros((_B, 2 * _DIM), _f32)
    hs = jnp.zeros((_B, _DIM), _f32)
    cs = jnp.zeros((_B, _DIM), _f32)
    for _ in range(6):
        g = (jnp.dot(q_star, sWihT_ref[...], preferred_element_type=_f32)
             + sbih_ref[...]
             + jnp.dot(hs, sWhhT_ref[...], preferred_element_type=_f32)
             + sbhh_ref[...])
        ig, fg, gg, og = (g[:, 0:16], g[:, 16:32], g[:, 32:48], g[:, 48:64])
        cs = jax.nn.sigmoid(fg) * cs + jax.nn.sigmoid(ig) * jnp.tanh(gg)
        hs = jax.nn.sigmoid(og) * jnp.tanh(cs)

        qb = jnp.dot(oh, hs, preferred_element_type=_f32,
                     precision=lax.Precision.HIGHEST)
        escore = jnp.sum(out * qb, axis=1, keepdims=True)
        masked = jnp.where(oh > 0.0, escore, -jnp.inf)
        emax = jnp.max(masked, axis=0, keepdims=True)
        emax = jnp.where(emax > -jnp.inf, emax, 0.0)
        emaxb = jnp.sum(oh * emax, axis=1, keepdims=True)
        ee = jnp.exp(escore - emaxb)
        denom = jnp.sum(oh * ee, axis=0, keepdims=True)
        denomb = jnp.sum(oh * denom, axis=1, keepdims=True)
        a = ee / (denomb + 1e-16)
        r = jnp.einsum('ng,nf->gf', oh, a * out,
                       preferred_element_type=_f32,
                       precision=lax.Precision.HIGHEST)
        q_star = jnp.concatenate([hs, r], axis=1)

    g = (jnp.dot(q_star, mWihT_ref[...], preferred_element_type=_f32)
         + mbih_ref[...] + mbhh_ref[...])
    ig, fg, gg, og = (g[:, 0:16], g[:, 16:32], g[:, 32:48], g[:, 48:64])
    cm = jax.nn.sigmoid(ig) * jnp.tanh(gg)
    hm = jax.nn.sigmoid(og) * jnp.tanh(cm)
    o = jax.nn.relu(jnp.dot(hm, W1T_ref[...], preferred_element_type=_f32)
                    + b1_ref[...])
    v_ref[...] = jnp.dot(o, W3T_ref[...], preferred_element_type=_f32) + b3_ref[...]
    hm_ref[...] = hm
    cm_ref[...] = cm


def _full(shape):
    return pl.BlockSpec(shape, lambda i: (0,) * len(shape))


def _rows(blk, width):
    return pl.BlockSpec((blk, width), lambda i: (i, 0))


def _pre_node(xp, w0t, b0, c0, c1):
    return pl.pallas_call(
        _pre_node_body,
        grid=(_N // _NBLK,),
        in_specs=[_rows(_NBLK, 8), _full((8, _DIM)), _full((1, _DIM)),
                  _rows(_NBLK, _DIM), _rows(_NBLK, _DIM)],
        out_specs=[_rows(_NBLK, _DIM), _rows(_NBLK, _DIM)],
        out_shape=[jax.ShapeDtypeStruct((_N, _DIM), _f32),
                   jax.ShapeDtypeStruct((_N, _DIM), _f32)],
    )(xp, w0t, b0, c0, c1)


def _tc_eh(eap, we1t, be1):
    return pl.pallas_call(
        _eh_body,
        grid=(_E_PAD // _EBLK,),
        in_specs=[_rows(_EBLK, _DIM), _full((_DIM, _DIM)), _full((1, _DIM))],
        out_specs=_rows(_EBLK, _DIM),
        out_shape=jax.ShapeDtypeStruct((_E_PAD, _DIM), _f32),
    )(eap, we1t, be1)


def _tc_msg(xj, eh, we2t, be2, rmat, smat):
    return pl.pallas_call(
        _msg_body,
        grid=(_E_PAD // _EBLK,),
        in_specs=[_rows(_EBLK, _DIM), _rows(_EBLK, _DIM),
                  _full((_DIM, _DIM * _DIM)), _full((1, _DIM * _DIM)),
                  _full((_DIM, _DIM * _DIM)), _full((_DIM * _DIM, _DIM))],
        out_specs=_rows(_EBLK, _DIM),
        out_shape=jax.ShapeDtypeStruct((_E_PAD, _DIM), _f32),
    )(xj, eh, we2t, be2, rmat, smat)


def _tc_gru(a0, a1, cinv, h, root, cb, wih, whh, bih, bhh):
    return pl.pallas_call(
        _gru_body,
        grid=(_N // _NBLK,),
        in_specs=[_rows(_NBLK, _DIM), _rows(_NBLK, _DIM), _rows(_NBLK, _DIM),
                  _rows(_NBLK, _DIM), _full((_DIM, _DIM)), _full((1, _DIM)),
                  _full((_DIM, 3 * _DIM)), _full((_DIM, 3 * _DIM)),
                  _full((1, 3 * _DIM)), _full((1, 3 * _DIM))],
        out_specs=_rows(_NBLK, _DIM),
        out_shape=jax.ShapeDtypeStruct((_N, _DIM), _f32),
    )(a0, a1, cinv, h, root, cb, wih, whh, bih, bhh)


def _tc_s2s(out, b2, sWihT, sWhhT, sbih, sbhh, mWihT, mWhhT, mbih, mbhh,
            W1T, b1, W3T, b3):
    return pl.pallas_call(
        _s2s_body,
        grid=(1,),
        in_specs=[_rows(_N, _DIM), _rows(_N, 1),
                  _full((2 * _DIM, 4 * _DIM)), _full((_DIM, 4 * _DIM)),
                  _full((1, 4 * _DIM)), _full((1, 4 * _DIM)),
                  _full((2 * _DIM, 4 * _DIM)), _full((_DIM, 4 * _DIM)),
                  _full((1, 4 * _DIM)), _full((1, 4 * _DIM)),
                  _full((_DIM, _DIM)), _full((1, _DIM)),
                  _full((_DIM, 1)), _full((1, 1))],
        out_specs=[_full((_B, 1)), _full((_B, _DIM)), _full((_B, _DIM))],
        out_shape=[jax.ShapeDtypeStruct((_B, 1), _f32),
                   jax.ShapeDtypeStruct((_B, _DIM), _f32),
                   jax.ShapeDtypeStruct((_B, _DIM), _f32)],
    )(out, b2, sWihT, sWhhT, sbih, sbhh, mWihT, mWhhT, mbih, mbhh,
      W1T, b1, W3T, b3)


def kernel(x, edge_index, edge_attr, batch, W0, b0, We1, be1, We2, be2, root,
           conv_bias, gWih, gWhh, gbih, gbhh, sWih, sWhh, sbih, sbhh,
           mWih, mWhh, mbih, mbhh, W1, b1, W3, b3):
    pad_e = _E_PAD - _E
    src = edge_index[0]
    dst = edge_index[1]
    srcp = jnp.concatenate(
        [src, jnp.zeros((pad_e,), jnp.int32)]).reshape(_E_PAD // _CH, _CH)
    dstp = jnp.concatenate(
        [dst, jnp.full((pad_e,), _N, jnp.int32)]).reshape(_E_PAD // _CH, _CH)
    eap = jnp.concatenate(
        [edge_attr, jnp.zeros((pad_e, edge_attr.shape[1]), _f32)], axis=0)
    xp = jnp.concatenate([x, jnp.zeros((_N, 8 - x.shape[1]), _f32)], axis=1)
    w0tp = jnp.concatenate(
        [W0.T, jnp.zeros((8 - x.shape[1], _DIM), _f32)], axis=0)
    zeros_tbl = jnp.zeros((_N_PAD, _DIM), _f32)
    ones_msg = jnp.ones((_E_PAD, _DIM), _f32)

    we2t = We2.T
    be2row = be2.reshape(1, _DIM * _DIM)
    lanes = jnp.arange(_DIM * _DIM, dtype=jnp.int32)
    dims = jnp.arange(_DIM, dtype=jnp.int32)
    rmat = (lanes[None, :] // _DIM == dims[:, None]).astype(_f32)
    smat = (lanes[:, None] % _DIM == dims[None, :]).astype(_f32)

    sc_scatter = _get_sc_scatter()
    sc_gather = _get_sc_gather()

    cntp = sc_scatter(ones_msg, dstp, zeros_tbl)
    out, cinv = _pre_node(xp, w0tp, b0.reshape(1, _DIM),
                          cntp[0, :_N], cntp[1, :_N])
    eh = _tc_eh(eap, We1.T, be1.reshape(1, _DIM))

    for _ in range(6):
        xj = sc_gather(out, srcp)
        msg = _tc_msg(xj, eh, we2t, be2row, rmat, smat)
        aggp = sc_scatter(msg, dstp, zeros_tbl)
        out = _tc_gru(aggp[0, :_N], aggp[1, :_N], cinv, out, root,
                      conv_bias.reshape(1, _DIM), gWih.T, gWhh.T,
                      gbih.reshape(1, 3 * _DIM), gbhh.reshape(1, 3 * _DIM))

    v, hm, cm = _tc_s2s(out, batch.reshape(_N, 1), sWih.T, sWhh.T,
                        sbih.reshape(1, 4 * _DIM), sbhh.reshape(1, 4 * _DIM),
                        mWih.T, mWhh.T, mbih.reshape(1, 4 * _DIM),
                        mbhh.reshape(1, 4 * _DIM), W1.T, b1.reshape(1, _DIM),
                        W3.T, b3.reshape(1, 1))
    return v[None], hm[None], cm[None]
